# baseline (speedup 1.0000x reference)
import functools

import jax
import jax.numpy as jnp
from jax.experimental import pallas as pl
from jax.experimental.pallas import tpu as pltpu

_B = 16384
_N = 1000
_TOPN = 50
_NPAD = 1024
_ROWS = 8


def _bitonic_desc(v, idx):
    rows, npad = v.shape
    lane = jax.lax.broadcasted_iota(jnp.int32, (rows, npad), 1)
    size = 2
    while size <= npad:
        d = size // 2
        while d >= 1:
            low = (lane & d) == 0
            desc = (lane & size) == 0
            take_big = desc == low
            pv = jnp.where(low, jnp.roll(v, -d, axis=1), jnp.roll(v, d, axis=1))
            pi = jnp.where(low, jnp.roll(idx, -d, axis=1), jnp.roll(idx, d, axis=1))
            pred = (v > pv) | ((v == pv) & (idx < pi))
            sel = pred == take_big
            v = jnp.where(sel, v, pv)
            idx = jnp.where(sel, idx, pi)
            d //= 2
        size *= 2
    return v, idx


def _sort_kernel(x_ref, idx_ref, conf_ref):
    v = x_ref[...]
    pad = jnp.full((_ROWS, _NPAD - _N), -jnp.inf, dtype=jnp.float32)
    v = jnp.concatenate([v, pad], axis=1)
    idx = jax.lax.broadcasted_iota(jnp.int32, (_ROWS, _NPAD), 1)
    v, idx = _bitonic_desc(v, idx)
    conf_ref[...] = v[:, :_N]
    idx_ref[...] = idx[:, :_TOPN]


@jax.jit
def kernel(x, label_ids):
    grid = (_B // _ROWS,)
    top_idx, conf = pl.pallas_call(
        _sort_kernel,
        grid=grid,
        in_specs=[pl.BlockSpec((_ROWS, _N), lambda i: (i, 0))],
        out_specs=[
            pl.BlockSpec((_ROWS, _TOPN), lambda i: (i, 0)),
            pl.BlockSpec((_ROWS, _N), lambda i: (i, 0)),
        ],
        out_shape=[
            jax.ShapeDtypeStruct((_B, _TOPN), jnp.int32),
            jax.ShapeDtypeStruct((_B, _N), jnp.float32),
        ],
        compiler_params=pltpu.CompilerParams(
            dimension_semantics=("parallel",),
        ),
    )(x)
    top_label = jnp.take(label_ids, top_idx)
    return (top_label, conf)


# speedup vs baseline: 1.3339x; 1.3339x over previous
import functools

import jax
import jax.numpy as jnp
from jax.experimental import pallas as pl
from jax.experimental.pallas import tpu as pltpu

_B = 16384
_N = 1000
_TOPN = 50
_NPAD = 1024
_ROWS = 32


def _bitonic_desc(v, idx):
    rows, npad = v.shape
    lane = jax.lax.broadcasted_iota(jnp.int32, (rows, npad), 1)
    size = 2
    while size <= npad:
        d = size // 2
        while d >= 1:
            low = (lane & d) == 0
            desc = (lane & size) == 0
            take_big = desc == low
            pv = jnp.where(low, jnp.roll(v, -d, axis=1), jnp.roll(v, d, axis=1))
            pi = jnp.where(low, jnp.roll(idx, -d, axis=1), jnp.roll(idx, d, axis=1))
            pred = (v > pv) | ((v == pv) & (idx < pi))
            sel = pred == take_big
            v = jnp.where(sel, v, pv)
            idx = jnp.where(sel, idx, pi)
            d //= 2
        size *= 2
    return v, idx


def _sort_kernel(x_ref, idx_ref, conf_ref):
    v = x_ref[...]
    pad = jnp.full((_ROWS, _NPAD - _N), -jnp.inf, dtype=jnp.float32)
    v = jnp.concatenate([v, pad], axis=1)
    idx = jax.lax.broadcasted_iota(jnp.int32, (_ROWS, _NPAD), 1)
    v, idx = _bitonic_desc(v, idx)
    conf_ref[...] = v[:, :_N]
    idx_ref[...] = idx[:, :_TOPN]


@jax.jit
def kernel(x, label_ids):
    grid = (_B // _ROWS,)
    top_idx, conf = pl.pallas_call(
        _sort_kernel,
        grid=grid,
        in_specs=[pl.BlockSpec((_ROWS, _N), lambda i: (i, 0))],
        out_specs=[
            pl.BlockSpec((_ROWS, _TOPN), lambda i: (i, 0)),
            pl.BlockSpec((_ROWS, _N), lambda i: (i, 0)),
        ],
        out_shape=[
            jax.ShapeDtypeStruct((_B, _TOPN), jnp.int32),
            jax.ShapeDtypeStruct((_B, _N), jnp.float32),
        ],
        compiler_params=pltpu.CompilerParams(
            dimension_semantics=("parallel",),
        ),
    )(x)
    top_label = jnp.take(label_ids, top_idx)
    return (top_label, conf)


# speedup vs baseline: 1.4324x; 1.0739x over previous
import functools

import numpy as np
import jax
import jax.numpy as jnp
from jax import lax
from jax.experimental import pallas as pl
from jax.experimental.pallas import tpu as pltpu
from jax.experimental.pallas import tpu_sc as plsc

_B = 16384
_N = 1000
_TOPN = 50
_P = 1024
_NV = _P // 16
_TOPP = 64
_MIN32 = np.int32(-(1 << 31))

_NW = 32
_ROWS_PER_W = _B // _NW

_mesh = plsc.VectorSubcoreMesh(core_axis_name="c", subcore_axis_name="s")


@functools.partial(
    pl.kernel,
    mesh=_mesh,
    out_type=[
        jax.ShapeDtypeStruct((_B, _TOPP), jnp.int32),
        jax.ShapeDtypeStruct((_B, _N), jnp.float32),
    ],
    scratch_types=[
        pltpu.VMEM((_P,), jnp.float32),
        pltpu.VMEM((_P,), jnp.int32),
        pltpu.VMEM((_P,), jnp.int32),
        pltpu.VMEM((_P,), jnp.int32),
        pltpu.VMEM((_P,), jnp.int32),
        pltpu.VMEM((256,), jnp.int32),
        pltpu.VMEM((16,), jnp.int32),
    ],
    compiler_params=pltpu.CompilerParams(
        needs_layout_passes=False, use_tc_tiling_on_sc=False),
)
def _sc_sort(x_hbm, oidx_hbm, oconf_hbm, vbuf, ka, ia, kb, ib, bins, tots):
    wid = lax.axis_index("s") * 2 + lax.axis_index("c")
    row0 = wid * _ROWS_PER_W

    lanes = lax.iota(jnp.int32, 16)
    zero = lanes * 0
    one = zero + 1
    ninf = zero.astype(jnp.float32) + jnp.float32(-jnp.inf)
    lane_is_last = lanes == 15

    def do_row(r, _):
        row = row0 + r
        vbuf[pl.ds(_P - 32, 16)] = ninf
        vbuf[pl.ds(_P - 16, 16)] = ninf
        pltpu.sync_copy(x_hbm.at[row], vbuf.at[pl.ds(0, _N)])

        def init_j(j, _):
            f = vbuf[pl.ds(j * 16, 16)]
            u = plsc.bitcast(f, jnp.int32)
            m = lax.shift_right_arithmetic(u, 31) | _MIN32
            ka[pl.ds(j * 16, 16)] = ~(u ^ m)
            ia[pl.ds(j * 16, 16)] = lanes + j * 16
            return 0
        lax.fori_loop(0, _NV, init_j, 0, unroll=4)

        def radix_pass(shift, src_k, src_i, dst_k, dst_i):
            def clr(t, _):
                bins[pl.ds(t * 16, 16)] = zero
                return 0
            lax.fori_loop(0, 16, clr, 0, unroll=4)

            def hist(j, _):
                k = src_k[pl.ds(j * 16, 16)]
                d = lax.shift_right_logical(k, shift) & 255
                plsc.addupdate_scatter(bins, [d], one)
                return 0
            lax.fori_loop(0, _NV, hist, 0, unroll=4)

            def scang(t, _):
                v = bins[pl.ds(t * 16, 16)]
                c = plsc.cumsum(v)
                bins[pl.ds(t * 16, 16)] = c - v
                plsc.store_scatter(tots, [zero + t], c, mask=lane_is_last)
                return 0
            lax.fori_loop(0, 16, scang, 0)
            tv = tots[...]
            tots[...] = plsc.cumsum(tv) - tv

            def addb(t, _):
                te = plsc.load_gather(tots, [zero + t])
                bins[pl.ds(t * 16, 16)] = bins[pl.ds(t * 16, 16)] + te
                return 0
            lax.fori_loop(0, 16, addb, 0)

            def perm(j, _):
                k = src_k[pl.ds(j * 16, 16)]
                d = lax.shift_right_logical(k, shift) & 255
                base = plsc.load_gather(bins, [d])
                sc, _unused = plsc.scan_count(d)
                ofs = base + sc - 1
                plsc.store_scatter(dst_k, [ofs], k)
                plsc.store_scatter(dst_i, [ofs], src_i[pl.ds(j * 16, 16)])
                plsc.addupdate_scatter(bins, [d], one)
                return 0
            lax.fori_loop(0, _NV, perm, 0)

        radix_pass(0, ka, ia, kb, ib)
        radix_pass(8, kb, ib, ka, ia)
        radix_pass(16, ka, ia, kb, ib)
        radix_pass(24, kb, ib, ka, ia)

        def fin_j(j, _):
            kd = ka[pl.ds(j * 16, 16)]
            k0 = ~kd
            m = (~lax.shift_right_arithmetic(k0, 31)) | _MIN32
            vbuf[pl.ds(j * 16, 16)] = plsc.bitcast(k0 ^ m, jnp.float32)
            return 0
        lax.fori_loop(0, _NV, fin_j, 0, unroll=4)

        pltpu.sync_copy(vbuf.at[pl.ds(0, _N)], oconf_hbm.at[row])
        pltpu.sync_copy(ia.at[pl.ds(0, _TOPP)], oidx_hbm.at[row])
        return 0

    lax.fori_loop(0, _ROWS_PER_W, do_row, 0)


@jax.jit
def kernel(x, label_ids):
    top_idx, conf = _sc_sort(x)
    top_label = jnp.take(label_ids, top_idx[:, :_TOPN])
    return (top_label, conf)
